# baseline (speedup 1.0000x reference)
import functools

import jax
import jax.numpy as jnp
from jax import lax
from jax.experimental import pallas as pl
from jax.experimental.pallas import tpu as pltpu
from jax.experimental.pallas import tpu_sc as plsc


_NEG_INF = float("-inf")
_NW = 32


def _row_pass(lab_ref, gt_ref, pb_ref, gb_ref, m4_ref, ls_ref, stats_ref):
    P, C = lab_ref.shape[1], lab_ref.shape[2]
    x = lab_ref[0]
    e = jnp.exp(x)
    s = jnp.sum(e, axis=1)
    lse = jnp.log(s)
    x0 = x[:, 0]
    gt = gt_ref[0, 0]
    pos = gt > 0
    posf = pos.astype(jnp.float32)
    pos_cnt = jnp.sum(posf)
    lse_pos = jnp.sum(jnp.where(pos, lse, 0.0))
    x0_neg = jnp.sum(jnp.where(pos, 0.0, x0))
    lsneg = jnp.where(pos, _NEG_INF, lse - x0)

    pb = pb_ref[0]
    gb = gb_ref[0]
    m4 = m4_ref[0]
    d = jnp.abs(pb - gb)
    sl = jnp.where(d < 1.0, 0.5 * d * d, d - 0.5)
    loc = jnp.sum(sl * m4)

    ls_ref[0, 0] = lsneg
    lane = lax.broadcasted_iota(jnp.int32, (128,), 0)
    stats_ref[0, 0] = jnp.where(
        lane == 0, pos_cnt,
        jnp.where(lane == 1, lse_pos,
                  jnp.where(lane == 2, loc,
                            jnp.where(lane == 3, x0_neg, 0.0))))


def _select(ls_ref, st_ref, gs_ref, npr_ref, out_ref):
    Brows, P = ls_ref.shape
    LS = ls_ref[...]
    K = lax.bitcast_convert_type(LS, jnp.int32)
    K = K ^ (jnp.int32(0x7FFFFFFF) & (K >> 31))
    st = st_ref[...]
    pos_cnt = st[:, 0:1]
    lse_pos = st[:, 1:2]
    loc_sum = st[:, 2:3]
    x0_neg = st[:, 3:4]
    g_all = jnp.sum(gs_ref[...])
    npr = npr_ref[0, 0]
    pos_i = pos_cnt.astype(jnp.int32)
    neg_i = P - pos_i
    k = jnp.minimum(npr * pos_i, neg_i)

    lo = jnp.full((Brows, 1), -2**31, jnp.int32)
    hi = jnp.full((Brows, 1), 2**31 - 1, jnp.int32)

    def it(_, lohi):
        lo, hi = lohi
        xr = lo ^ hi
        mid = (lo & hi) + (xr >> 1) + (xr & 1)
        cnt = jnp.sum((K >= mid).astype(jnp.int32), axis=1, keepdims=True)
        ge = cnt >= k
        return jnp.where(ge, mid, lo), jnp.where(ge, hi, mid - 1)

    lo, hi = lax.fori_loop(0, 32, it, (lo, hi))
    T = lo
    tb = T ^ (jnp.int32(0x7FFFFFFF) & (T >> 31))
    thr = lax.bitcast_convert_type(tb, jnp.float32)
    gtm = K > T
    cnt_gt = jnp.sum(gtm.astype(jnp.int32), axis=1, keepdims=True)
    sum_gt = jnp.sum(jnp.where(gtm, LS, 0.0), axis=1, keepdims=True)
    topk = jnp.where(
        k > 0, sum_gt + (k - cnt_gt).astype(jnp.float32) * thr, 0.0)

    conf_total = (jnp.sum(lse_pos) - g_all + jnp.sum(x0_neg)
                  + jnp.sum(topk))
    loc_total = jnp.sum(loc_sum)
    denom = jnp.sum(pos_cnt) + 1e-7
    lane = lax.broadcasted_iota(jnp.int32, (128,), 0)
    out_ref[0] = jnp.where(
        lane == 0, loc_total / denom,
        jnp.where(lane == 1, conf_total / denom, 0.0))


def _make_sc_gather(n_total, n_classes):
    chunk = n_total // _NW
    assert chunk * _NW == n_total and chunk % 8 == 0
    nrows = (chunk + 127) // 128
    mesh = plsc.VectorSubcoreMesh(core_axis_name="c", subcore_axis_name="s")

    @functools.partial(
        pl.kernel, mesh=mesh,
        out_type=jax.ShapeDtypeStruct((_NW, 16), jnp.float32),
        scratch_types=[
            pltpu.VMEM((nrows * 128,), jnp.int32),
            pltpu.VMEM((nrows, 128), jnp.int32),
            pltpu.VMEM((nrows, 128), jnp.float32),
            pltpu.VMEM((16,), jnp.float32),
            pltpu.SemaphoreType.DMA,
        ],
    )
    def sc_gather(gt_hbm, lab_hbm, out_hbm, gt_v, idx_v, val_v, acc_v, sem):
        wid = lax.axis_index("s") * 2 + lax.axis_index("c")
        base = wid * chunk
        pltpu.sync_copy(gt_hbm.at[pl.ds(base, chunk)],
                        gt_v.at[pl.ds(0, chunk)])
        lane16 = lax.broadcasted_iota(jnp.int32, (16,), 0)

        def idx_row(i, carry):
            for l in range(8):
                off = i * 128 + l * 16
                valid = (off + lane16) < chunk
                gtv = gt_v[pl.ds(off, 16)]
                flat = (base + off + lane16) * n_classes + gtv
                idx_v[i, pl.ds(l * 16, 16)] = jnp.where(valid, flat, 0)
            return carry

        lax.fori_loop(0, nrows, idx_row, 0)

        def fire(j, carry):
            pltpu.async_copy(lab_hbm.at[idx_v.at[j]], val_v.at[j], sem)
            return carry

        lax.fori_loop(0, nrows, fire, 0)

        def drain(j, carry):
            pltpu.make_async_copy(lab_hbm.at[idx_v.at[j]], val_v.at[j],
                                  sem).wait()
            return carry

        lax.fori_loop(0, nrows, drain, 0)

        def accum(i, acc):
            for l in range(8):
                off = i * 128 + l * 16
                valid = (off + lane16) < chunk
                v = val_v[i, pl.ds(l * 16, 16)]
                acc = acc + jnp.where(valid, v, 0.0)
            return acc

        acc = lax.fori_loop(0, nrows, accum, jnp.zeros((16,), jnp.float32))
        acc_v[...] = acc
        pltpu.sync_copy(acc_v, out_hbm.at[wid])

    return sc_gather


def _ssd_loss(predict_bboxes, predict_labels, gt_bboxes, gt_labels, npr):
    B, P, C = predict_labels.shape
    gt3 = gt_labels.reshape(B, 1, P)
    pad4 = ((4 * P + 127) // 128) * 128
    rows4 = pad4 // 128
    flat_pad = lambda a: jnp.pad(
        a.reshape(B, -1), ((0, 0), (0, pad4 - 4 * P))).reshape(B, rows4, 128)
    pb4 = flat_pad(predict_bboxes)
    gb4 = flat_pad(gt_bboxes)
    posf = (gt_labels > 0).astype(jnp.float32)
    m4 = flat_pad(jnp.repeat(posf, 4, axis=1))

    gsum = _make_sc_gather(B * P, C)(
        gt_labels.reshape(B * P), predict_labels.reshape(B * P * C))

    ls, stats = pl.pallas_call(
        _row_pass,
        grid=(B,),
        in_specs=[
            pl.BlockSpec((1, P, C), lambda b: (b, 0, 0)),
            pl.BlockSpec((1, 1, P), lambda b: (b, 0, 0)),
            pl.BlockSpec((1, rows4, 128), lambda b: (b, 0, 0)),
            pl.BlockSpec((1, rows4, 128), lambda b: (b, 0, 0)),
            pl.BlockSpec((1, rows4, 128), lambda b: (b, 0, 0)),
        ],
        out_specs=[
            pl.BlockSpec((1, 1, P), lambda b: (b, 0, 0)),
            pl.BlockSpec((1, 1, 128), lambda b: (b, 0, 0)),
        ],
        out_shape=[
            jax.ShapeDtypeStruct((B, 1, P), jnp.float32),
            jax.ShapeDtypeStruct((B, 1, 128), jnp.float32),
        ],
    )(predict_labels, gt3, pb4, gb4, m4)
    out = pl.pallas_call(
        _select,
        in_specs=[
            pl.BlockSpec((B, P), lambda: (0, 0)),
            pl.BlockSpec((B, 128), lambda: (0, 0)),
            pl.BlockSpec((_NW // 8, 128), lambda: (0, 0)),
            pl.BlockSpec((1, 1), lambda: (0, 0)),
        ],
        out_specs=pl.BlockSpec((1, 128), lambda: (0, 0)),
        out_shape=jax.ShapeDtypeStruct((1, 128), jnp.float32),
    )(ls.reshape(B, P), stats.reshape(B, 128),
      gsum.reshape(_NW // 8, 128), npr.reshape(1, 1))
    return out[0, 0], out[0, 1]


def kernel(predict_bboxes, predict_labels, gt_bboxes, gt_labels,
           neg_pos_ratio):
    npr = jnp.asarray(neg_pos_ratio, jnp.int32)
    return _ssd_loss(predict_bboxes, predict_labels, gt_bboxes, gt_labels,
                     npr)


# speedup vs baseline: 4.4077x; 4.4077x over previous
import jax
import jax.numpy as jnp
from jax import lax
from jax.experimental import pallas as pl


_NEG_INF = float("-inf")


def _row_pass(lab_ref, gt_ref, pb_ref, gb_ref, ls_ref, stats_ref):
    P, C = lab_ref.shape[1], lab_ref.shape[2]
    x = lab_ref[0]
    e = jnp.exp(x)
    s = jnp.sum(e, axis=1)
    lse = jnp.log(s)
    gt = gt_ref[0, 0]
    cio = lax.broadcasted_iota(jnp.int32, (1, C), 1)
    g = jnp.sum(jnp.where(cio == gt[:, None], x, 0.0), axis=1)
    ce = lse - g
    pos = gt > 0
    posf = pos.astype(jnp.float32)
    pos_cnt = jnp.sum(posf)
    ce_pos = jnp.sum(jnp.where(pos, ce, 0.0))
    lsneg = jnp.where(pos, _NEG_INF, ce)

    pb = pb_ref[0]
    gb = gb_ref[0]
    d = jnp.abs(pb - gb)
    sl = jnp.where(d < 1.0, 0.5 * d * d, d - 0.5)
    loc = jnp.sum(sl * posf[None, :])

    ls_ref[0, 0] = lsneg
    lane = lax.broadcasted_iota(jnp.int32, (128,), 0)
    stats_ref[0, 0] = jnp.where(
        lane == 0, pos_cnt,
        jnp.where(lane == 1, ce_pos, jnp.where(lane == 2, loc, 0.0)))


def _select(ls_ref, st_ref, npr_ref, out_ref):
    Brows, P = ls_ref.shape
    LS = ls_ref[...]
    K = lax.bitcast_convert_type(LS, jnp.int32)
    K = K ^ (jnp.int32(0x7FFFFFFF) & (K >> 31))
    st = st_ref[...]
    pos_cnt = st[:, 0:1]
    ce_pos = st[:, 1:2]
    loc_sum = st[:, 2:3]
    npr = npr_ref[0, 0]
    pos_i = pos_cnt.astype(jnp.int32)
    neg_i = P - pos_i
    k = jnp.minimum(npr * pos_i, neg_i)

    lo = jnp.full((Brows, 1), -2**31, jnp.int32)
    hi = jnp.full((Brows, 1), 2**31 - 1, jnp.int32)

    def it(_, lohi):
        lo, hi = lohi
        xr = lo ^ hi
        mid = (lo & hi) + (xr >> 1) + (xr & 1)
        cnt = jnp.sum((K >= mid).astype(jnp.int32), axis=1, keepdims=True)
        ge = cnt >= k
        return jnp.where(ge, mid, lo), jnp.where(ge, hi, mid - 1)

    lo, hi = lax.fori_loop(0, 32, it, (lo, hi))
    T = lo
    tb = T ^ (jnp.int32(0x7FFFFFFF) & (T >> 31))
    thr = lax.bitcast_convert_type(tb, jnp.float32)
    gtm = K > T
    cnt_gt = jnp.sum(gtm.astype(jnp.int32), axis=1, keepdims=True)
    sum_gt = jnp.sum(jnp.where(gtm, LS, 0.0), axis=1, keepdims=True)
    topk = jnp.where(
        k > 0, sum_gt + (k - cnt_gt).astype(jnp.float32) * thr, 0.0)

    conf_total = jnp.sum(ce_pos + topk)
    loc_total = jnp.sum(loc_sum)
    denom = jnp.sum(pos_cnt) + 1e-7
    lane = lax.broadcasted_iota(jnp.int32, (128,), 0)
    out_ref[0] = jnp.where(
        lane == 0, loc_total / denom,
        jnp.where(lane == 1, conf_total / denom, 0.0))


def _ssd_loss(predict_bboxes, predict_labels, gt_bboxes, gt_labels, npr):
    B, P, C = predict_labels.shape
    pb = jnp.transpose(predict_bboxes, (0, 2, 1))
    gb = jnp.transpose(gt_bboxes, (0, 2, 1))
    gt3 = gt_labels.reshape(B, 1, P)
    ls, stats = pl.pallas_call(
        _row_pass,
        grid=(B,),
        in_specs=[
            pl.BlockSpec((1, P, C), lambda b: (b, 0, 0)),
            pl.BlockSpec((1, 1, P), lambda b: (b, 0, 0)),
            pl.BlockSpec((1, 4, P), lambda b: (b, 0, 0)),
            pl.BlockSpec((1, 4, P), lambda b: (b, 0, 0)),
        ],
        out_specs=[
            pl.BlockSpec((1, 1, P), lambda b: (b, 0, 0)),
            pl.BlockSpec((1, 1, 128), lambda b: (b, 0, 0)),
        ],
        out_shape=[
            jax.ShapeDtypeStruct((B, 1, P), jnp.float32),
            jax.ShapeDtypeStruct((B, 1, 128), jnp.float32),
        ],
    )(predict_labels, gt3, pb, gb)
    out = pl.pallas_call(
        _select,
        in_specs=[
            pl.BlockSpec((B, P), lambda: (0, 0)),
            pl.BlockSpec((B, 128), lambda: (0, 0)),
            pl.BlockSpec((1, 1), lambda: (0, 0)),
        ],
        out_specs=pl.BlockSpec((1, 128), lambda: (0, 0)),
        out_shape=jax.ShapeDtypeStruct((1, 128), jnp.float32),
    )(ls.reshape(B, P), stats.reshape(B, 128), npr.reshape(1, 1))
    return out[0, 0], out[0, 1]


def kernel(predict_bboxes, predict_labels, gt_bboxes, gt_labels,
           neg_pos_ratio):
    npr = jnp.asarray(neg_pos_ratio, jnp.int32)
    return _ssd_loss(predict_bboxes, predict_labels, gt_bboxes, gt_labels,
                     npr)


# speedup vs baseline: 4.5412x; 1.0303x over previous
import jax
import jax.numpy as jnp
from jax import lax
from jax.experimental import pallas as pl


_NEG_INF = float("-inf")


def _row_pass(lab_ref, gt_ref, pb_ref, gb_ref, ls_ref, stats_ref):
    P, C = lab_ref.shape[1], lab_ref.shape[2]
    x = lab_ref[0]
    e = jnp.exp(x)
    s = jnp.sum(e, axis=1)
    lse = jnp.log(s)
    gt = gt_ref[0, 0]
    gtb = gt.astype(jnp.bfloat16)[:, None]
    ones_row = jnp.ones((1, 128), jnp.bfloat16)
    bc = jax.lax.dot_general(gtb, ones_row, (((1,), (0,)), ((), ())),
                             preferred_element_type=jnp.float32)
    cio = lax.broadcasted_iota(jnp.int32, (1, C), 1).astype(jnp.float32)
    g = jnp.sum(jnp.where(bc[:, :C] == cio, x, 0.0), axis=1)
    ce = lse - g
    pos = gt > 0
    posf = pos.astype(jnp.float32)
    pos_cnt = jnp.sum(posf)
    ce_pos = jnp.sum(jnp.where(pos, ce, 0.0))
    lsneg = jnp.where(pos, _NEG_INF, ce)

    pb = pb_ref[0]
    gb = gb_ref[0]
    d = jnp.abs(pb - gb)
    sl = jnp.where(d < 1.0, 0.5 * d * d, d - 0.5)
    loc = jnp.sum(sl * posf[None, :])

    ls_ref[0, 0] = lsneg
    lane = lax.broadcasted_iota(jnp.int32, (128,), 0)
    stats_ref[0, 0] = jnp.where(
        lane == 0, pos_cnt,
        jnp.where(lane == 1, ce_pos, jnp.where(lane == 2, loc, 0.0)))


def _select(ls_ref, st_ref, npr_ref, out_ref):
    Brows, P = ls_ref.shape
    LS = ls_ref[...]
    K = lax.bitcast_convert_type(LS, jnp.int32)
    K = K ^ (jnp.int32(0x7FFFFFFF) & (K >> 31))
    st = st_ref[...]
    pos_cnt = st[:, 0:1]
    ce_pos = st[:, 1:2]
    loc_sum = st[:, 2:3]
    npr = npr_ref[0, 0]
    pos_i = pos_cnt.astype(jnp.int32)
    neg_i = P - pos_i
    k = jnp.minimum(npr * pos_i, neg_i)

    lo = jnp.full((Brows, 1), -2**31, jnp.int32)
    hi = jnp.full((Brows, 1), 2**31 - 1, jnp.int32)

    def it(_, lohi):
        lo, hi = lohi
        xr = lo ^ hi
        mid = (lo & hi) + (xr >> 1) + (xr & 1)
        cnt = jnp.sum((K >= mid).astype(jnp.int32), axis=1, keepdims=True)
        ge = cnt >= k
        return jnp.where(ge, mid, lo), jnp.where(ge, hi, mid - 1)

    lo, hi = lax.fori_loop(0, 32, it, (lo, hi))
    T = lo
    tb = T ^ (jnp.int32(0x7FFFFFFF) & (T >> 31))
    thr = lax.bitcast_convert_type(tb, jnp.float32)
    gtm = K > T
    cnt_gt = jnp.sum(gtm.astype(jnp.int32), axis=1, keepdims=True)
    sum_gt = jnp.sum(jnp.where(gtm, LS, 0.0), axis=1, keepdims=True)
    topk = jnp.where(
        k > 0, sum_gt + (k - cnt_gt).astype(jnp.float32) * thr, 0.0)

    conf_total = jnp.sum(ce_pos + topk)
    loc_total = jnp.sum(loc_sum)
    denom = jnp.sum(pos_cnt) + 1e-7
    lane = lax.broadcasted_iota(jnp.int32, (128,), 0)
    out_ref[0] = jnp.where(
        lane == 0, loc_total / denom,
        jnp.where(lane == 1, conf_total / denom, 0.0))


def _ssd_loss(predict_bboxes, predict_labels, gt_bboxes, gt_labels, npr):
    B, P, C = predict_labels.shape
    pb = jnp.transpose(predict_bboxes, (0, 2, 1))
    gb = jnp.transpose(gt_bboxes, (0, 2, 1))
    gt3 = gt_labels.reshape(B, 1, P)
    ls, stats = pl.pallas_call(
        _row_pass,
        grid=(B,),
        in_specs=[
            pl.BlockSpec((1, P, C), lambda b: (b, 0, 0)),
            pl.BlockSpec((1, 1, P), lambda b: (b, 0, 0)),
            pl.BlockSpec((1, 4, P), lambda b: (b, 0, 0)),
            pl.BlockSpec((1, 4, P), lambda b: (b, 0, 0)),
        ],
        out_specs=[
            pl.BlockSpec((1, 1, P), lambda b: (b, 0, 0)),
            pl.BlockSpec((1, 1, 128), lambda b: (b, 0, 0)),
        ],
        out_shape=[
            jax.ShapeDtypeStruct((B, 1, P), jnp.float32),
            jax.ShapeDtypeStruct((B, 1, 128), jnp.float32),
        ],
    )(predict_labels, gt3, pb, gb)
    out = pl.pallas_call(
        _select,
        in_specs=[
            pl.BlockSpec((B, P), lambda: (0, 0)),
            pl.BlockSpec((B, 128), lambda: (0, 0)),
            pl.BlockSpec((1, 1), lambda: (0, 0)),
        ],
        out_specs=pl.BlockSpec((1, 128), lambda: (0, 0)),
        out_shape=jax.ShapeDtypeStruct((1, 128), jnp.float32),
    )(ls.reshape(B, P), stats.reshape(B, 128), npr.reshape(1, 1))
    return out[0, 0], out[0, 1]


def kernel(predict_bboxes, predict_labels, gt_bboxes, gt_labels,
           neg_pos_ratio):
    npr = jnp.asarray(neg_pos_ratio, jnp.int32)
    return _ssd_loss(predict_bboxes, predict_labels, gt_bboxes, gt_labels,
                     npr)
